# baseline (speedup 1.0000x reference)
import jax
import jax.numpy as jnp
from jax.experimental import pallas as pl
from jax.experimental.pallas import tpu as pltpu

_F8 = jnp.float8_e4m3fn


def _p1_kernel(x_ref, w1_ref, out_ref):
    out_ref[...] = jnp.dot(x_ref[...], w1_ref[...],
                           preferred_element_type=jnp.float32)


def _pass1_kernel(adj_ref, big_ref, p1_ref, w2_ref, b1_ref,
                  p2_ref, adj8_ref, d_ref):
    a = adj_ref[...]
    h = jnp.dot(a, p1_ref[...], preferred_element_type=jnp.float32)
    h = jnp.maximum(h + b1_ref[...], 0.0)
    p2_ref[...] = jnp.dot(h, w2_ref[...], preferred_element_type=jnp.float32)
    adj8_ref[...] = a.astype(_F8)
    d_ref[...] = jnp.sum(big_ref[...], axis=1, keepdims=True)


def _pass2_kernel(adj8_ref, big_ref, p2_ref, w3_ref, b2_ref, ones_ref,
                  p3_ref, d_ref):
    a = adj8_ref[...].astype(jnp.float32)
    h = jnp.dot(a, p2_ref[...], preferred_element_type=jnp.float32)
    h = jnp.maximum(h + b2_ref[...], 0.0)
    p3_ref[...] = jnp.dot(h, w3_ref[...], preferred_element_type=jnp.float32)
    d_ref[...] = jnp.dot(big_ref[...], ones_ref[...],
                         preferred_element_type=jnp.float32)


def _pass3_kernel(adj8_ref, big_ref, p3_ref, b3_ref, ones_ref,
                  h_ref, d_ref):
    a = adj8_ref[...].astype(jnp.float32)
    h = jnp.dot(a, p3_ref[...], preferred_element_type=jnp.float32)
    h_ref[...] = h + b3_ref[...]
    d_ref[...] = jnp.dot(big_ref[...], ones_ref[...],
                         preferred_element_type=jnp.float32)


def _epi_kernel(h_ref, x_ref, d_ref, out_ref):
    h = jnp.where(d_ref[...] == 0.0, x_ref[...], h_ref[...])
    out_ref[...] = jnp.maximum(h, 0.0)


def kernel(x, adj, Adj, W1, b1, W2, b2, W3, b3):
    n, nfeat = x.shape
    nmid1 = W1.shape[1]
    nmid2 = W2.shape[1]
    nhid = W3.shape[1]

    tm1 = 200
    tm23 = 400
    br1 = (tm1 * 3) // 5
    br2 = tm23 // 5
    br3 = tm23 // 5
    r1 = br1 * (n // tm1)
    r2 = br2 * (n // tm23)
    ones = jnp.ones((n, 1), jnp.float32)

    p1 = pl.pallas_call(
        _p1_kernel,
        out_shape=jax.ShapeDtypeStruct((n, nmid1), jnp.float32),
    )(x, W1)

    p2, adj8, d1 = pl.pallas_call(
        _pass1_kernel,
        grid=(n // tm1,),
        in_specs=[
            pl.BlockSpec((tm1, n), lambda i: (i, 0)),
            pl.BlockSpec((br1, n), lambda i: (i, 0)),
            pl.BlockSpec((n, nmid1), lambda i: (0, 0)),
            pl.BlockSpec((nmid1, nmid2), lambda i: (0, 0)),
            pl.BlockSpec((1, nmid1), lambda i: (0, 0)),
        ],
        out_specs=[
            pl.BlockSpec((tm1, nmid2), lambda i: (i, 0)),
            pl.BlockSpec((tm1, n), lambda i: (i, 0)),
            pl.BlockSpec((br1, 1), lambda i: (i, 0)),
        ],
        out_shape=[
            jax.ShapeDtypeStruct((n, nmid2), jnp.float32),
            jax.ShapeDtypeStruct((n, n), _F8),
            jax.ShapeDtypeStruct((r1, 1), jnp.float32),
        ],
        compiler_params=pltpu.CompilerParams(
            dimension_semantics=("arbitrary",)),
    )(adj, Adj, p1, W2, b1.reshape(1, -1))

    off2 = r1 // br2

    p3, d2 = pl.pallas_call(
        _pass2_kernel,
        grid=(n // tm23,),
        in_specs=[
            pl.BlockSpec((tm23, n), lambda i: (i, 0)),
            pl.BlockSpec((br2, n), lambda i: (i + off2, 0)),
            pl.BlockSpec((n, nmid2), lambda i: (0, 0)),
            pl.BlockSpec((nmid2, nhid), lambda i: (0, 0)),
            pl.BlockSpec((1, nmid2), lambda i: (0, 0)),
            pl.BlockSpec((n, 1), lambda i: (0, 0)),
        ],
        out_specs=[
            pl.BlockSpec((tm23, nhid), lambda i: (i, 0)),
            pl.BlockSpec((br2, 1), lambda i: (i, 0)),
        ],
        out_shape=[
            jax.ShapeDtypeStruct((n, nhid), jnp.float32),
            jax.ShapeDtypeStruct((r2, 1), jnp.float32),
        ],
        compiler_params=pltpu.CompilerParams(
            dimension_semantics=("arbitrary",)),
    )(adj8, Adj, p2, W3, b2.reshape(1, -1), ones)

    off3 = (r1 + r2) // br3

    h3, d3 = pl.pallas_call(
        _pass3_kernel,
        grid=(n // tm23,),
        in_specs=[
            pl.BlockSpec((tm23, n), lambda i: (i, 0)),
            pl.BlockSpec((br3, n), lambda i: (i + off3, 0)),
            pl.BlockSpec((n, nhid), lambda i: (0, 0)),
            pl.BlockSpec((1, nhid), lambda i: (0, 0)),
            pl.BlockSpec((n, 1), lambda i: (0, 0)),
        ],
        out_specs=[
            pl.BlockSpec((tm23, nhid), lambda i: (i, 0)),
            pl.BlockSpec((br3, 1), lambda i: (i, 0)),
        ],
        out_shape=[
            jax.ShapeDtypeStruct((n, nhid), jnp.float32),
            jax.ShapeDtypeStruct((n - r1 - r2, 1), jnp.float32),
        ],
        compiler_params=pltpu.CompilerParams(
            dimension_semantics=("arbitrary",)),
    )(adj8, Adj, p3, b3.reshape(1, -1), ones)

    d = jnp.concatenate([d1, d2, d3], axis=0)

    out = pl.pallas_call(
        _epi_kernel,
        grid=(n // tm23,),
        in_specs=[
            pl.BlockSpec((tm23, nhid), lambda i: (i, 0)),
            pl.BlockSpec((tm23, nfeat), lambda i: (i, 0)),
            pl.BlockSpec((tm23, 1), lambda i: (i, 0)),
        ],
        out_specs=pl.BlockSpec((tm23, nhid), lambda i: (i, 0)),
        out_shape=jax.ShapeDtypeStruct((n, nhid), jnp.float32),
        compiler_params=pltpu.CompilerParams(
            dimension_semantics=("arbitrary",)),
    )(h3, x, d)

    return out


# speedup vs baseline: 1.0800x; 1.0800x over previous
import jax
import jax.numpy as jnp
from jax.experimental import pallas as pl
from jax.experimental.pallas import tpu as pltpu

_F8 = jnp.float8_e4m3fn


def _p1_kernel(x_ref, w1_ref, out_ref):
    out_ref[...] = jnp.dot(x_ref[...], w1_ref[...],
                           preferred_element_type=jnp.float32)


def _pass1_kernel(adj_ref, big_ref, p1_ref, w2_ref, b1_ref,
                  p2_ref, adj8_ref, d_ref):
    a = adj_ref[...]
    h = jnp.dot(a, p1_ref[...], preferred_element_type=jnp.float32)
    h = jnp.maximum(h + b1_ref[...], 0.0)
    p2_ref[...] = jnp.dot(h, w2_ref[...], preferred_element_type=jnp.float32)
    adj8_ref[...] = a.astype(_F8)
    d_ref[...] = jnp.sum(big_ref[...], axis=1, keepdims=True)


def _pass2_kernel(adj8_ref, p2_ref, w3_ref, b2_ref, p3_ref):
    a = adj8_ref[...].astype(jnp.bfloat16)
    p2 = p2_ref[...].astype(jnp.bfloat16)
    h = jnp.dot(a, p2, preferred_element_type=jnp.float32)
    h = jnp.maximum(h + b2_ref[...], 0.0)
    p3_ref[...] = jnp.dot(h, w3_ref[...], preferred_element_type=jnp.float32)


def _pass3_kernel(adj8_ref, p3_ref, x_ref, b3_ref, d_ref, out_ref):
    a = adj8_ref[...].astype(jnp.bfloat16)
    p3 = p3_ref[...].astype(jnp.bfloat16)
    h = jnp.dot(a, p3, preferred_element_type=jnp.float32)
    h = h + b3_ref[...]
    h = jnp.where(d_ref[...] == 0.0, x_ref[...], h)
    out_ref[...] = jnp.maximum(h, 0.0)


def kernel(x, adj, Adj, W1, b1, W2, b2, W3, b3):
    n, nfeat = x.shape
    nmid1 = W1.shape[1]
    nmid2 = W2.shape[1]
    nhid = W3.shape[1]

    tm1 = 200 if n % 200 == 0 else n
    tm23 = 400 if n % 400 == 0 else n

    p1 = pl.pallas_call(
        _p1_kernel,
        out_shape=jax.ShapeDtypeStruct((n, nmid1), jnp.float32),
    )(x, W1)

    p2, adj8, d = pl.pallas_call(
        _pass1_kernel,
        grid=(n // tm1,),
        in_specs=[
            pl.BlockSpec((tm1, n), lambda i: (i, 0)),
            pl.BlockSpec((tm1, n), lambda i: (i, 0)),
            pl.BlockSpec((n, nmid1), lambda i: (0, 0)),
            pl.BlockSpec((nmid1, nmid2), lambda i: (0, 0)),
            pl.BlockSpec((1, nmid1), lambda i: (0, 0)),
        ],
        out_specs=[
            pl.BlockSpec((tm1, nmid2), lambda i: (i, 0)),
            pl.BlockSpec((tm1, n), lambda i: (i, 0)),
            pl.BlockSpec((tm1, 1), lambda i: (i, 0)),
        ],
        out_shape=[
            jax.ShapeDtypeStruct((n, nmid2), jnp.float32),
            jax.ShapeDtypeStruct((n, n), _F8),
            jax.ShapeDtypeStruct((n, 1), jnp.float32),
        ],
        compiler_params=pltpu.CompilerParams(
            dimension_semantics=("arbitrary",)),
    )(adj, Adj, p1, W2, b1.reshape(1, -1))

    p3 = pl.pallas_call(
        _pass2_kernel,
        grid=(n // tm23,),
        in_specs=[
            pl.BlockSpec((tm23, n), lambda i: (i, 0)),
            pl.BlockSpec((n, nmid2), lambda i: (0, 0)),
            pl.BlockSpec((nmid2, nhid), lambda i: (0, 0)),
            pl.BlockSpec((1, nmid2), lambda i: (0, 0)),
        ],
        out_specs=pl.BlockSpec((tm23, nhid), lambda i: (i, 0)),
        out_shape=jax.ShapeDtypeStruct((n, nhid), jnp.float32),
        compiler_params=pltpu.CompilerParams(
            dimension_semantics=("arbitrary",)),
    )(adj8, p2, W3, b2.reshape(1, -1))

    out = pl.pallas_call(
        _pass3_kernel,
        grid=(n // tm23,),
        in_specs=[
            pl.BlockSpec((tm23, n), lambda i: (i, 0)),
            pl.BlockSpec((n, nhid), lambda i: (0, 0)),
            pl.BlockSpec((tm23, nfeat), lambda i: (i, 0)),
            pl.BlockSpec((1, nhid), lambda i: (0, 0)),
            pl.BlockSpec((tm23, 1), lambda i: (i, 0)),
        ],
        out_specs=pl.BlockSpec((tm23, nhid), lambda i: (i, 0)),
        out_shape=jax.ShapeDtypeStruct((n, nhid), jnp.float32),
        compiler_params=pltpu.CompilerParams(
            dimension_semantics=("arbitrary",)),
    )(adj8, p3, x, b3.reshape(1, -1), d)

    return out
